# baseline (speedup 1.0000x reference)
import functools

import jax
import jax.numpy as jnp
from jax import lax
from jax.experimental import pallas as pl
from jax.experimental.pallas import tpu as pltpu
from jax.experimental.pallas import tpu_sc as plsc

_info = plsc.get_sparse_core_info()
_NC = _info.num_cores
_NS = _info.num_subcores
_L = _info.num_lanes
_NW = _NC * _NS

_CHUNK = 32
_NBUF = 4


@functools.lru_cache(maxsize=None)
def _make(q, c, d, bn):
    epw = bn * q // _NW
    assert epw % (4 * _CHUNK) == 0 and _CHUNK % _L == 0
    assert q & (q - 1) == 0 and _L % q == 0
    bn_per_w = bn // _NW
    n_chunks = epw // _CHUNK
    vecs_per_chunk = _CHUNK // _L
    assert (n_chunks - 4) % 4 == 0 and n_chunks >= 8

    mesh = plsc.VectorSubcoreMesh(core_axis_name="c", subcore_axis_name="s")

    @functools.partial(
        pl.kernel,
        mesh=mesh,
        out_type=jax.ShapeDtypeStruct((q * bn, d), jnp.float32),
        scratch_types=[
            pltpu.VMEM((epw,), jnp.int32),
            pltpu.VMEM((n_chunks, _CHUNK), jnp.int32),
            pltpu.VMEM((n_chunks, _CHUNK), jnp.int32),
            pltpu.VMEM((_CHUNK, d), jnp.float32),
            pltpu.VMEM((_CHUNK, d), jnp.float32),
            pltpu.VMEM((_CHUNK, d), jnp.float32),
            pltpu.VMEM((_CHUNK, d), jnp.float32),
            pltpu.SemaphoreType.DMA,
            pltpu.SemaphoreType.DMA,
        ],
    )
    def k(idx_hbm, cb_hbm, out_hbm, idx_v, gidx_v, oidx_v,
          buf0, buf1, buf2, buf3, gsem, wsem):
        wid = lax.axis_index("s") * _NC + lax.axis_index("c")
        bufs = (buf0, buf1, buf2, buf3)
        obase = wid * bn_per_w

        pltpu.sync_copy(idx_hbm.at[pl.ds(wid * epw, epw)], idx_v)

        lanes = lax.iota(jnp.int32, _L)
        qv = lanes & (q - 1)
        cb_bias = qv * c
        out_bias = qv * bn + obase + lax.shift_right_logical(
            lanes, q.bit_length() - 1)

        def fill(j):
            for i in range(vecs_per_chunk):
                base = j * vecs_per_chunk + i
                vec = idx_v[pl.ds(base * _L, _L)]
                gidx_v[j, pl.ds(i * _L, _L)] = vec + cb_bias
                oidx_v[j, pl.ds(i * _L, _L)] = out_bias + base * (_L // q)

        def gather_start(j, b):
            pltpu.async_copy(cb_hbm.at[gidx_v.at[j]], bufs[b], gsem)

        def gather_wait(j, b):
            pltpu.make_async_copy(cb_hbm.at[gidx_v.at[j]], bufs[b], gsem).wait()

        def scatter_start(j, b):
            pltpu.async_copy(bufs[b], out_hbm.at[oidx_v.at[j]], wsem)

        def scatter_wait(j, b):
            pltpu.make_async_copy(bufs[b], out_hbm.at[oidx_v.at[j]], wsem).wait()

        fill(0)
        gather_start(0, 0)
        fill(1)
        gather_start(1, 1)
        fill(2)
        gather_wait(0, 0)
        scatter_start(0, 0)
        gather_start(2, 2)
        fill(3)
        gather_wait(1, 1)
        scatter_start(1, 1)
        gather_start(3, 3)

        def ring(gi, carry):
            for db in range(4):
                j = 4 * gi + 2 + db
                b = (2 + db) % _NBUF
                gather_wait(j, b)
                scatter_start(j, b)
                scatter_wait(j - 2, db % _NBUF)
                fill(j + 2)
                gather_start(j + 2, db % _NBUF)
            return carry
        lax.fori_loop(0, (n_chunks - 4) // 4, ring, 0)

        for db in range(2):
            j = n_chunks - 2 + db
            gather_wait(j, j % _NBUF)
            scatter_start(j, j % _NBUF)
        for db in range(4):
            j = n_chunks - 4 + db
            scatter_wait(j, j % _NBUF)

    return k


def kernel(indices, codebooks):
    q, c, d = codebooks.shape
    idx_flat = indices.reshape(-1)
    bn = idx_flat.size // q
    cb_flat = codebooks.reshape(q * c, d)
    out = _make(q, c, d, bn)(idx_flat, cb_flat)
    return out.reshape((q,) + indices.shape[:-1] + (d,))


# speedup vs baseline: 1.0623x; 1.0623x over previous
import functools

import jax
import jax.numpy as jnp
from jax import lax
from jax.experimental import pallas as pl
from jax.experimental.pallas import tpu as pltpu
from jax.experimental.pallas import tpu_sc as plsc

_info = plsc.get_sparse_core_info()
_NC = _info.num_cores
_NS = _info.num_subcores
_L = _info.num_lanes
_NW = _NC * _NS

_CHUNK = 64
_NBUF = 4


@functools.lru_cache(maxsize=None)
def _make(q, c, d, bn):
    epw = bn * q // _NW
    assert epw % (4 * _CHUNK) == 0 and _CHUNK % _L == 0
    assert q & (q - 1) == 0 and _L % q == 0
    bn_per_w = bn // _NW
    n_chunks = epw // _CHUNK
    vecs_per_chunk = _CHUNK // _L
    assert (n_chunks - 4) % 4 == 0 and n_chunks >= 8

    mesh = plsc.VectorSubcoreMesh(core_axis_name="c", subcore_axis_name="s")

    @functools.partial(
        pl.kernel,
        mesh=mesh,
        out_type=jax.ShapeDtypeStruct((q * bn, d), jnp.float32),
        scratch_types=[
            pltpu.VMEM((epw,), jnp.int32),
            pltpu.VMEM((n_chunks, _CHUNK), jnp.int32),
            pltpu.VMEM((n_chunks, _CHUNK), jnp.int32),
            pltpu.VMEM((_CHUNK, d), jnp.float32),
            pltpu.VMEM((_CHUNK, d), jnp.float32),
            pltpu.VMEM((_CHUNK, d), jnp.float32),
            pltpu.VMEM((_CHUNK, d), jnp.float32),
            pltpu.SemaphoreType.DMA,
            pltpu.SemaphoreType.DMA,
        ],
    )
    def k(idx_hbm, cb_hbm, out_hbm, idx_v, gidx_v, oidx_v,
          buf0, buf1, buf2, buf3, gsem, wsem):
        wid = lax.axis_index("s") * _NC + lax.axis_index("c")
        bufs = (buf0, buf1, buf2, buf3)
        obase = wid * bn_per_w

        pltpu.sync_copy(idx_hbm.at[pl.ds(wid * epw, epw)], idx_v)

        lanes = lax.iota(jnp.int32, _L)
        qv = lanes & (q - 1)
        cb_bias = qv * c
        out_bias = qv * bn + obase + lax.shift_right_logical(
            lanes, q.bit_length() - 1)

        def fill(j):
            for i in range(vecs_per_chunk):
                base = j * vecs_per_chunk + i
                vec = idx_v[pl.ds(base * _L, _L)]
                gidx_v[j, pl.ds(i * _L, _L)] = vec + cb_bias
                oidx_v[j, pl.ds(i * _L, _L)] = out_bias + base * (_L // q)

        def gather_start(j, b):
            pltpu.async_copy(cb_hbm.at[gidx_v.at[j]], bufs[b], gsem)

        def gather_wait(j, b):
            pltpu.make_async_copy(cb_hbm.at[gidx_v.at[j]], bufs[b], gsem).wait()

        def scatter_start(j, b):
            pltpu.async_copy(bufs[b], out_hbm.at[oidx_v.at[j]], wsem)

        def scatter_wait(j, b):
            pltpu.make_async_copy(bufs[b], out_hbm.at[oidx_v.at[j]], wsem).wait()

        fill(0)
        gather_start(0, 0)
        fill(1)
        gather_start(1, 1)
        fill(2)
        gather_wait(0, 0)
        scatter_start(0, 0)
        gather_start(2, 2)
        fill(3)
        gather_wait(1, 1)
        scatter_start(1, 1)
        gather_start(3, 3)

        def ring(gi, carry):
            for db in range(4):
                j = 4 * gi + 2 + db
                b = (2 + db) % _NBUF
                gather_wait(j, b)
                scatter_start(j, b)
                scatter_wait(j - 2, db % _NBUF)
                fill(j + 2)
                gather_start(j + 2, db % _NBUF)
            return carry
        lax.fori_loop(0, (n_chunks - 4) // 4, ring, 0)

        for db in range(2):
            j = n_chunks - 2 + db
            gather_wait(j, j % _NBUF)
            scatter_start(j, j % _NBUF)
        for db in range(4):
            j = n_chunks - 4 + db
            scatter_wait(j, j % _NBUF)

    return k


def kernel(indices, codebooks):
    q, c, d = codebooks.shape
    idx_flat = indices.reshape(-1)
    bn = idx_flat.size // q
    cb_flat = codebooks.reshape(q * c, d)
    out = _make(q, c, d, bn)(idx_flat, cb_flat)
    return out.reshape((q,) + indices.shape[:-1] + (d,))
